# baseline (speedup 1.0000x reference)
import functools

import jax
import jax.numpy as jnp
from jax import lax
from jax.experimental import pallas as pl
from jax.experimental.pallas import tpu as pltpu
from jax.experimental.pallas import tpu_sc as plsc

_NUM_FEATURES = 26
_VOCAB = 128
_BATCH = 16384
_MAX_NORM = 1.0

_TABLE_ROWS = _NUM_FEATURES * _VOCAB
_WIDTH = _NUM_FEATURES * _VOCAB

_NC, _NS, _L = 2, 16, 16
_NW = _NC * _NS
_BATCH_PER_W = _BATCH // _NW
_QROWS = 128
_NQ = _BATCH_PER_W // _QROWS
_NPAIR = _NUM_FEATURES // 2


def _scale_body(*refs):
    o_ref = refs[-1]
    for i in range(_NUM_FEATURES):
        rows = refs[i][...]
        norm = jnp.sqrt(jnp.sum(rows * rows, axis=1, keepdims=True))
        scale = jnp.minimum(1.0, _MAX_NORM / jnp.maximum(norm, 1e-7))
        o_ref[pl.ds(i * _VOCAB, _VOCAB), :] = rows * scale


_scale_call = pl.pallas_call(
    _scale_body,
    out_shape=jax.ShapeDtypeStruct((_TABLE_ROWS, _VOCAB), jnp.float32),
)


def _sc_body(xt_hbm, table_hbm, out_hbm, xtv, tbl_a, tbl_b, cb0, cb1,
             tsem_a, tsem_b, osem0, osem1):
    tbls = (tbl_a, tbl_b)
    tsems = (tsem_a, tsem_b)
    cbs = (cb0, cb1)
    osems = (osem0, osem1)
    wid = lax.axis_index("s") * _NC + lax.axis_index("c")
    brow = wid * _BATCH_PER_W

    pltpu.sync_copy(xt_hbm.at[pl.ds(0, _NUM_FEATURES), pl.ds(brow, _BATCH_PER_W)],
                    xtv)
    pltpu.sync_copy(table_hbm.at[pl.ds(0, _VOCAB)], tbl_a)

    def _drain_out(q):
        pltpu.make_async_copy(
            cbs[q % 2],
            out_hbm.at[pl.ds(brow, _QROWS), pl.ds(0, _VOCAB)],
            osems[q % 2]).wait()

    def _assemble(f, tbl, first_feature, p):
        for q in range(_NQ):
            cb = cbs[q % 2]
            if first_feature and q < 2:
                @pl.when(p > 0)
                def _():
                    _drain_out(q)
            else:
                _drain_out(q)

            def grp(g, carry):
                jbase = q * _QROWS + g * _L
                sv = xtv[pl.ds(f, 1), pl.ds(jbase, _L)]
                for r in range(_L):
                    s = sv[0, r]
                    row = g * _L + r
                    for c in range(_VOCAB // _L):
                        cb[pl.ds(row, 1), pl.ds(c * _L, _L)] = (
                            tbl[pl.ds(s, 1), pl.ds(c * _L, _L)])
                return carry

            lax.fori_loop(0, _QROWS // _L, grp, 0)
            pltpu.async_copy(
                cb,
                out_hbm.at[pl.ds(brow + q * _QROWS, _QROWS),
                           pl.ds(f * _VOCAB, _VOCAB)],
                osems[q % 2])

    def pair_body(p, carry):
        f0 = p * 2
        f1 = f0 + 1
        cp_b = pltpu.async_copy(
            table_hbm.at[pl.ds(f1 * _VOCAB, _VOCAB)], tbl_b, tsem_b)
        _assemble(f0, tbl_a, True, p)
        cp_b.wait()
        fnext = jnp.minimum(f1 + 1, _NUM_FEATURES - 1)
        cp_a = pltpu.async_copy(
            table_hbm.at[pl.ds(fnext * _VOCAB, _VOCAB)], tbl_a, tsem_a)
        _assemble(f1, tbl_b, False, p)
        cp_a.wait()
        return carry

    lax.fori_loop(0, _NPAIR, pair_body, 0)
    _drain_out(0)
    _drain_out(1)


@functools.cache
def _make_sc_assemble():
    mesh = plsc.VectorSubcoreMesh(core_axis_name="c", subcore_axis_name="s")
    return pl.kernel(
        _sc_body,
        mesh=mesh,
        out_type=jax.ShapeDtypeStruct((_BATCH, _WIDTH), jnp.float32),
        scratch_types=[
            pltpu.VMEM((_NUM_FEATURES, _BATCH_PER_W), jnp.int32),
            pltpu.VMEM((_VOCAB, _VOCAB), jnp.float32),
            pltpu.VMEM((_VOCAB, _VOCAB), jnp.float32),
            pltpu.VMEM((_QROWS, _VOCAB), jnp.float32),
            pltpu.VMEM((_QROWS, _VOCAB), jnp.float32),
            pltpu.SemaphoreType.DMA,
            pltpu.SemaphoreType.DMA,
            pltpu.SemaphoreType.DMA,
            pltpu.SemaphoreType.DMA,
        ],
    )


def kernel(x, W0, W1, W2, W3, W4, W5, W6, W7, W8, W9, W10, W11, W12, W13,
           W14, W15, W16, W17, W18, W19, W20, W21, W22, W23, W24, W25):
    Ws = [W0, W1, W2, W3, W4, W5, W6, W7, W8, W9, W10, W11, W12, W13,
          W14, W15, W16, W17, W18, W19, W20, W21, W22, W23, W24, W25]
    scaled = _scale_call(*Ws)
    xt = jnp.transpose(x.astype(jnp.int32))
    return _make_sc_assemble()(xt, scaled)


# speedup vs baseline: 1.8794x; 1.8794x over previous
import functools

import jax
import jax.numpy as jnp
from jax import lax
from jax.experimental import pallas as pl
from jax.experimental.pallas import tpu as pltpu
from jax.experimental.pallas import tpu_sc as plsc

_NUM_FEATURES = 26
_VOCAB = 128
_BATCH = 16384
_MAX_NORM = 1.0

_ROWS = _BATCH * _NUM_FEATURES
_TABLE_ROWS = _NUM_FEATURES * _VOCAB
_WIDTH = _NUM_FEATURES * _VOCAB

_NC, _NS, _L = 2, 16, 16
_NW = _NC * _NS
_PER_W = _ROWS // _NW
_BATCH_PER_W = _BATCH // _NW
_CHUNK = 104
_BROWS = _CHUNK // _NUM_FEATURES
_NBUF = 8
_CHUNKS_PER_W = _PER_W // _CHUNK
_OUTER = _CHUNKS_PER_W // _NBUF


def _scale_body(*refs):
    x_ref = refs[0]
    o_ref = refs[-2]
    g_ref = refs[-1]
    for i in range(_NUM_FEATURES):
        rows = refs[i + 1][...]
        norm = jnp.sqrt(jnp.sum(rows * rows, axis=1, keepdims=True))
        scale = jnp.minimum(1.0, _MAX_NORM / jnp.maximum(norm, 1e-7))
        o_ref[pl.ds(i * _VOCAB, _VOCAB), :] = rows * scale
    offs = lax.broadcasted_iota(jnp.int32, (_BATCH, _NUM_FEATURES), 1) * _VOCAB
    g_ref[...] = x_ref[...] + offs


_scale_call = pl.pallas_call(
    _scale_body,
    out_shape=(
        jax.ShapeDtypeStruct((_TABLE_ROWS, _VOCAB), jnp.float32),
        jax.ShapeDtypeStruct((_BATCH, _NUM_FEATURES), jnp.int32),
    ),
)


def _sc_gather_body(gidx_hbm, table_hbm, out_hbm, gidxv,
                    b0, b1, b2, b3, b4, b5, b6, b7,
                    s0, s1, s2, s3, s4, s5, s6, s7,
                    t0, t1, t2, t3, t4, t5, t6, t7):
    bufs = (b0, b1, b2, b3, b4, b5, b6, b7)
    sems = (s0, s1, s2, s3, s4, s5, s6, s7)
    osems = (t0, t1, t2, t3, t4, t5, t6, t7)
    wid = lax.axis_index("s") * _NC + lax.axis_index("c")
    brow = wid * _BATCH_PER_W

    pltpu.sync_copy(gidx_hbm.at[pl.ds(wid * _CHUNKS_PER_W, _CHUNKS_PER_W)],
                    gidxv)

    def _drain_out(b):
        pltpu.make_async_copy(
            bufs[b].reshape(_BROWS, _WIDTH),
            out_hbm.at[pl.ds(brow, _BROWS)], osems[b]).wait()

    def chunk_body(p, carry):
        copies = []
        for b in range(_NBUF):
            @pl.when(p > 0)
            def _():
                _drain_out(b)

            c = p * _NBUF + b
            copies.append(
                pltpu.async_copy(
                    table_hbm.at[gidxv.at[c]],
                    bufs[b],
                    sems[b],
                )
            )
        for b in range(_NBUF):
            c = p * _NBUF + b
            copies[b].wait()
            pltpu.async_copy(
                bufs[b].reshape(_BROWS, _WIDTH),
                out_hbm.at[pl.ds(brow + c * _BROWS, _BROWS)],
                osems[b])
        return carry

    lax.fori_loop(0, _OUTER, chunk_body, 0)
    for b in range(_NBUF):
        _drain_out(b)


@functools.cache
def _make_sc_gather():
    mesh = plsc.VectorSubcoreMesh(core_axis_name="c", subcore_axis_name="s")
    return pl.kernel(
        _sc_gather_body,
        mesh=mesh,
        out_type=jax.ShapeDtypeStruct((_BATCH, _WIDTH), jnp.float32),
        scratch_types=[
            pltpu.VMEM((_CHUNKS_PER_W, _CHUNK), jnp.int32),
        ] + [pltpu.VMEM((_CHUNK, _VOCAB), jnp.float32)] * _NBUF
          + [pltpu.SemaphoreType.DMA] * (2 * _NBUF),
    )


def kernel(x, W0, W1, W2, W3, W4, W5, W6, W7, W8, W9, W10, W11, W12, W13,
           W14, W15, W16, W17, W18, W19, W20, W21, W22, W23, W24, W25):
    Ws = [W0, W1, W2, W3, W4, W5, W6, W7, W8, W9, W10, W11, W12, W13,
          W14, W15, W16, W17, W18, W19, W20, W21, W22, W23, W24, W25]
    scaled, gidx = _scale_call(x.astype(jnp.int32), *Ws)
    gidx2 = gidx.reshape(_ROWS // _CHUNK, _CHUNK)
    return _make_sc_gather()(gidx2, scaled)


# speedup vs baseline: 2.0118x; 1.0705x over previous
import functools

import jax
import jax.numpy as jnp
from jax import lax
from jax.experimental import pallas as pl
from jax.experimental.pallas import tpu as pltpu
from jax.experimental.pallas import tpu_sc as plsc

_NUM_FEATURES = 26
_VOCAB = 128
_BATCH = 16384
_MAX_NORM = 1.0

_ROWS = _BATCH * _NUM_FEATURES
_TABLE_ROWS = _NUM_FEATURES * _VOCAB
_WIDTH = _NUM_FEATURES * _VOCAB

_NC, _NS, _L = 2, 16, 16
_NW = _NC * _NS
_PER_W = _ROWS // _NW
_BATCH_PER_W = _BATCH // _NW
_CHUNK = 104
_BROWS = _CHUNK // _NUM_FEATURES
_NBUF = 8
_CHUNKS_PER_W = _PER_W // _CHUNK
_OUTER = _CHUNKS_PER_W // _NBUF


def _scale_body(*refs):
    o_ref = refs[-1]
    for i in range(_NUM_FEATURES):
        rows = refs[i][...]
        norm = jnp.sqrt(jnp.sum(rows * rows, axis=1, keepdims=True))
        scale = jnp.minimum(1.0, _MAX_NORM / jnp.maximum(norm, 1e-7))
        o_ref[pl.ds(i * _VOCAB, _VOCAB), :] = rows * scale


_scale_call = pl.pallas_call(
    _scale_body,
    out_shape=jax.ShapeDtypeStruct((_TABLE_ROWS, _VOCAB), jnp.float32),
)


def _sc_gather_body(x_hbm, table_hbm, out_hbm, idxv,
                    b0, b1, b2, b3, b4, b5, b6, b7,
                    s0, s1, s2, s3, s4, s5, s6, s7,
                    t0, t1, t2, t3, t4, t5, t6, t7):
    bufs = (b0, b1, b2, b3, b4, b5, b6, b7)
    sems = (s0, s1, s2, s3, s4, s5, s6, s7)
    osems = (t0, t1, t2, t3, t4, t5, t6, t7)
    wid = lax.axis_index("s") * _NC + lax.axis_index("c")
    base = wid * _PER_W
    brow = wid * _BATCH_PER_W

    pltpu.sync_copy(x_hbm.at[pl.ds(base, _PER_W)], idxv)

    lanes = lax.iota(jnp.int32, _L)

    def idx_body(j, carry):
        r = j * _L
        off = lax.rem(base + r + lanes, _NUM_FEATURES) * _VOCAB
        idxv[pl.ds(r, _L)] = idxv[pl.ds(r, _L)] + off
        return carry

    lax.fori_loop(0, _PER_W // _L, idx_body, 0, unroll=4)

    def _drain_out(b):
        pltpu.make_async_copy(
            bufs[b].reshape(_BROWS, _WIDTH),
            out_hbm.at[pl.ds(brow, _BROWS)], osems[b]).wait()

    def chunk_body(p, carry):
        copies = []
        for b in range(_NBUF):
            @pl.when(p > 0)
            def _():
                _drain_out(b)

            c = p * _NBUF + b
            copies.append(
                pltpu.async_copy(
                    table_hbm.at[idxv.at[pl.ds(c * _CHUNK, _CHUNK)]],
                    bufs[b],
                    sems[b],
                )
            )
        for b in range(_NBUF):
            c = p * _NBUF + b
            copies[b].wait()
            pltpu.async_copy(
                bufs[b].reshape(_BROWS, _WIDTH),
                out_hbm.at[pl.ds(brow + c * _BROWS, _BROWS)],
                osems[b])
        return carry

    lax.fori_loop(0, _OUTER, chunk_body, 0)
    for b in range(_NBUF):
        _drain_out(b)


@functools.cache
def _make_sc_gather():
    mesh = plsc.VectorSubcoreMesh(core_axis_name="c", subcore_axis_name="s")
    return pl.kernel(
        _sc_gather_body,
        mesh=mesh,
        out_type=jax.ShapeDtypeStruct((_BATCH, _WIDTH), jnp.float32),
        scratch_types=[
            pltpu.VMEM((_PER_W,), jnp.int32),
        ] + [pltpu.VMEM((_CHUNK, _VOCAB), jnp.float32)] * _NBUF
          + [pltpu.SemaphoreType.DMA] * (2 * _NBUF),
    )


def kernel(x, W0, W1, W2, W3, W4, W5, W6, W7, W8, W9, W10, W11, W12, W13,
           W14, W15, W16, W17, W18, W19, W20, W21, W22, W23, W24, W25):
    Ws = [W0, W1, W2, W3, W4, W5, W6, W7, W8, W9, W10, W11, W12, W13,
          W14, W15, W16, W17, W18, W19, W20, W21, W22, W23, W24, W25]
    scaled = _scale_call(*Ws)
    x_flat = x.astype(jnp.int32).reshape(-1)
    return _make_sc_gather()(x_flat, scaled)


# speedup vs baseline: 2.0278x; 1.0080x over previous
import functools

import jax
import jax.numpy as jnp
from jax import lax
from jax.experimental import pallas as pl
from jax.experimental.pallas import tpu as pltpu
from jax.experimental.pallas import tpu_sc as plsc

_NUM_FEATURES = 26
_VOCAB = 128
_BATCH = 16384
_MAX_NORM = 1.0

_ROWS = _BATCH * _NUM_FEATURES
_TABLE_ROWS = _NUM_FEATURES * _VOCAB
_WIDTH = _NUM_FEATURES * _VOCAB

_NC, _NS, _L = 2, 16, 16
_NW = _NC * _NS
_PER_W = _ROWS // _NW
_BATCH_PER_W = _BATCH // _NW
_CHUNK = 104
_BROWS = _CHUNK // _NUM_FEATURES
_NBUF = 8
_CHUNKS_PER_W = _PER_W // _CHUNK
_OUTER = _CHUNKS_PER_W // _NBUF


def _scale_body(*refs):
    o_ref = refs[-1]
    for i in range(_NUM_FEATURES):
        rows = refs[i][...]
        norm = jnp.sqrt(jnp.sum(rows * rows, axis=1, keepdims=True))
        scale = jnp.minimum(1.0, _MAX_NORM / jnp.maximum(norm, 1e-7))
        o_ref[pl.ds(i * _VOCAB, _VOCAB), :] = rows * scale


_scale_call = pl.pallas_call(
    _scale_body,
    out_shape=jax.ShapeDtypeStruct((_TABLE_ROWS, _VOCAB), jnp.float32),
)


def _sc_gather_body(x_hbm, table_hbm, out_hbm, idxv, *bs):
    bufs = bs[:_NBUF]
    sems = bs[_NBUF:2 * _NBUF]
    osems = bs[2 * _NBUF:3 * _NBUF]
    wid = lax.axis_index("s") * _NC + lax.axis_index("c")
    base = wid * _PER_W
    brow = wid * _BATCH_PER_W

    pltpu.sync_copy(x_hbm.at[pl.ds(base, _PER_W)], idxv)

    lanes = lax.iota(jnp.int32, _L)

    def idx_body(j, carry):
        r = j * _L
        off = lax.rem(base + r + lanes, _NUM_FEATURES) * _VOCAB
        idxv[pl.ds(r, _L)] = idxv[pl.ds(r, _L)] + off
        return carry

    _FIRST = _NBUF * _CHUNK // _L
    lax.fori_loop(0, _FIRST, idx_body, 0, unroll=4)

    def _fire_gather(c, b):
        return pltpu.async_copy(
            table_hbm.at[idxv.at[pl.ds(c * _CHUNK, _CHUNK)]],
            bufs[b],
            sems[b],
        )

    def _fire_out(c, b):
        return pltpu.async_copy(
            bufs[b].reshape(_BROWS, _WIDTH),
            out_hbm.at[pl.ds(brow + c * _BROWS, _BROWS)],
            osems[b])

    def _drain_out(b):
        pltpu.make_async_copy(
            bufs[b].reshape(_BROWS, _WIDTH),
            out_hbm.at[pl.ds(brow, _BROWS)], osems[b]).wait()

    pre = [_fire_gather(b, b) for b in range(_NBUF)]

    lax.fori_loop(_FIRST, _PER_W // _L, idx_body, 0, unroll=4)

    for b in range(_NBUF):
        pre[b].wait()
        _fire_out(b, b)

    def chunk_body(p, carry):
        copies = []
        for b in range(_NBUF):
            _drain_out(b)
            copies.append(_fire_gather(p * _NBUF + b, b))
        for b in range(_NBUF):
            copies[b].wait()
            _fire_out(p * _NBUF + b, b)
        return carry

    lax.fori_loop(1, _OUTER, chunk_body, 0)
    for b in range(_NBUF):
        _drain_out(b)


@functools.cache
def _make_sc_gather():
    mesh = plsc.VectorSubcoreMesh(core_axis_name="c", subcore_axis_name="s")
    return pl.kernel(
        _sc_gather_body,
        mesh=mesh,
        out_type=jax.ShapeDtypeStruct((_BATCH, _WIDTH), jnp.float32),
        scratch_types=[
            pltpu.VMEM((_PER_W,), jnp.int32),
        ] + [pltpu.VMEM((_CHUNK, _VOCAB), jnp.float32)] * _NBUF
          + [pltpu.SemaphoreType.DMA] * (2 * _NBUF),
    )


def kernel(x, W0, W1, W2, W3, W4, W5, W6, W7, W8, W9, W10, W11, W12, W13,
           W14, W15, W16, W17, W18, W19, W20, W21, W22, W23, W24, W25):
    Ws = [W0, W1, W2, W3, W4, W5, W6, W7, W8, W9, W10, W11, W12, W13,
          W14, W15, W16, W17, W18, W19, W20, W21, W22, W23, W24, W25]
    scaled = _scale_call(*Ws)
    x_flat = x.astype(jnp.int32).reshape(-1)
    return _make_sc_gather()(x_flat, scaled)
